# baseline (speedup 1.0000x reference)
import jax
import jax.numpy as jnp
from jax import lax
from jax.experimental import pallas as pl
from jax.experimental.pallas import tpu as pltpu
from jax.experimental.pallas import tpu_sc as plsc

_MASK_PCT = 0.8
_STRIPE = 512


def _make_field_idx(bs: int):
    n_masked = int(_MASK_PCT * bs)
    jkey = jax.random.key(42)
    j = jax.random.randint(jkey, (n_masked,), 0, 5, dtype=jnp.int32)
    return jnp.concatenate([j, -jnp.ones((bs - n_masked,), dtype=jnp.int32)])


def _sc_vec_mask(j_hbm, age_hbm, gen_hbm, o_age, o_gen,
                 jv, av, gv, oav, ogv):
    s = lax.axis_index("s")
    row0 = s * _STRIPE

    sl = pl.ds(row0, _STRIPE)
    pltpu.sync_copy(j_hbm.at[sl], jv)
    pltpu.sync_copy(age_hbm.at[sl], av)
    pltpu.sync_copy(gen_hbm.at[sl], gv)

    def body(i, carry):
        v = pl.ds(i * 16, 16)
        jj = jv[v]
        z = jnp.zeros((16,), jnp.float32)
        oav[v] = jnp.where(jj != 3, av[v], z)
        ogv[v] = jnp.where(jj != 4, gv[v], z)
        return carry

    lax.fori_loop(0, _STRIPE // 16, body, 0)

    pltpu.sync_copy(oav, o_age.at[sl])
    pltpu.sync_copy(ogv, o_gen.at[sl])


def _tc_mask_kernel(j_ref, dgb_ref, prb_ref, odb_ref, dgp_ref, prp_ref,
                    odp_ref, o_dgb, o_prb, o_odb, o_dgp, o_prp, o_odp):
    j = j_ref[...]
    keep0 = (j != 0).astype(jnp.float32)[:, :, None]
    keep1 = (j != 1).astype(jnp.float32)[:, :, None]
    keep2 = (j != 2).astype(jnp.float32)[:, :, None]
    o_dgb[...] = dgb_ref[...] * keep0
    o_dgp[...] = dgp_ref[...] * keep0
    o_prb[...] = prb_ref[...] * keep1
    o_prp[...] = prp_ref[...] * keep1
    o_odb[...] = odb_ref[...] * keep2
    o_odp[...] = odp_ref[...] * keep2


def kernel(x_dg_bin, x_prod_bin, x_odb_bin, x_dg_pe, x_prod_pe, x_odb_pe,
           x_age, x_gender):
    B, D = x_dg_bin.shape
    R = B // 128
    j = _make_field_idx(B)

    half = B // 2
    half_t = jax.ShapeDtypeStruct((half,), jnp.float32)
    mesh = plsc.VectorSubcoreMesh(core_axis_name="c", subcore_axis_name="s",
                                  num_cores=1)
    sc_fn = pl.kernel(
        _sc_vec_mask,
        out_type=[half_t, half_t],
        mesh=mesh,
        scratch_types=[
            pltpu.VMEM((_STRIPE,), jnp.int32),
            pltpu.VMEM((_STRIPE,), jnp.float32),
            pltpu.VMEM((_STRIPE,), jnp.float32),
            pltpu.VMEM((_STRIPE,), jnp.float32),
            pltpu.VMEM((_STRIPE,), jnp.float32),
        ],
    )
    o_age0, o_gen0 = sc_fn(j[:half], x_age[:half], x_gender[:half])
    o_age1, o_gen1 = sc_fn(j[half:], x_age[half:], x_gender[half:])
    o_age = jnp.concatenate([o_age0, o_age1])
    o_gen = jnp.concatenate([o_gen0, o_gen1])

    jp = j.reshape(R, 128)
    bm = 32
    grid = (R // bm,)
    big3 = [x.reshape(R, 128, D) for x in
            (x_dg_bin, x_prod_bin, x_odb_bin, x_dg_pe, x_prod_pe, x_odb_pe)]
    bigs = pl.BlockSpec((bm, 128, D), lambda i: (i, 0, 0))
    vec = pl.BlockSpec((bm, 128), lambda i: (i, 0))
    big3_t = jax.ShapeDtypeStruct((R, 128, D), jnp.float32)

    o_dgb, o_prb, o_odb, o_dgp, o_prp, o_odp = pl.pallas_call(
        _tc_mask_kernel,
        grid=grid,
        in_specs=[vec] + [bigs] * 6,
        out_specs=[bigs] * 6,
        out_shape=[big3_t] * 6,
    )(jp, *big3)

    return (o_dgb.reshape(B, D), o_prb.reshape(B, D), o_odb.reshape(B, D),
            o_dgp.reshape(B, D), o_prp.reshape(B, D), o_odp.reshape(B, D),
            o_age, o_gen)
